# Initial kernel scaffold; baseline (speedup 1.0000x reference)
import jax
import jax.numpy as jnp
from jax.experimental import pallas as pl


def kernel(inputs, embedding_matrix):
    raise NotImplementedError("write your pallas kernel here")


# speedup vs baseline: 1.4587x; 1.4587x over previous
import functools

import jax
import jax.numpy as jnp
from jax import lax
from jax.experimental import pallas as pl
from jax.experimental.pallas import tpu as pltpu
from jax.experimental.pallas import tpu_sc as plsc

_NC = 2
_NS = 16
_NW = _NC * _NS

_IDX_W = 128
_K = 8
_CHUNK = _K * _IDX_W


def _sc_gather(table, idx2d, embed_dim):
    n_rows = idx2d.shape[0]
    rows_per_w = n_rows // _NW
    n_chunks = rows_per_w // _K

    mesh = plsc.VectorSubcoreMesh(core_axis_name="c", subcore_axis_name="s")

    @functools.partial(
        pl.kernel,
        out_type=jax.ShapeDtypeStruct((n_rows, _IDX_W, embed_dim), jnp.float32),
        mesh=mesh,
        scratch_types=[
            pltpu.VMEM((_K, _IDX_W), jnp.int32),
            pltpu.VMEM((_K, _IDX_W, embed_dim), jnp.float32),
            pltpu.SemaphoreType.DMA,
        ],
        compiler_params=pltpu.CompilerParams(use_tc_tiling_on_sc=False),
    )
    def k(table_hbm, idx_hbm, out_hbm, idx_v, rows_v, sem):
        wid = lax.axis_index("s") * _NC + lax.axis_index("c")
        base = wid * rows_per_w

        def step(g, carry):
            row0 = base + g * _K
            pltpu.sync_copy(idx_hbm.at[pl.ds(row0, _K)], idx_v)
            copies = [
                pltpu.async_copy(table_hbm.at[idx_v.at[j]], rows_v.at[j], sem)
                for j in range(_K)
            ]
            for c in copies:
                c.wait()
            pltpu.sync_copy(rows_v, out_hbm.at[pl.ds(row0, _K)])
            return carry

        lax.fori_loop(0, n_chunks, step, 0)

    return k(table, idx2d)


def kernel(inputs, embedding_matrix):
    batch, seq = inputs.shape
    vocab, embed_dim = embedding_matrix.shape
    n = batch * seq
    idx2d = inputs.astype(jnp.int32).reshape(n // _IDX_W, _IDX_W)
    out = _sc_gather(embedding_matrix, idx2d, embed_dim)
    return out.reshape(batch, seq, embed_dim)


# speedup vs baseline: 1.4947x; 1.0246x over previous
import functools

import jax
import jax.numpy as jnp
from jax import lax
from jax.experimental import pallas as pl
from jax.experimental.pallas import tpu as pltpu
from jax.experimental.pallas import tpu_sc as plsc

_NC = 2
_NS = 16
_NW = _NC * _NS

_IDX_W = 128
_K = 10


def _sc_gather(table, idx2d, embed_dim):
    n_rows = idx2d.shape[0]
    rows_per_w = n_rows // _NW
    n_chunks = rows_per_w // _K

    mesh = plsc.VectorSubcoreMesh(core_axis_name="c", subcore_axis_name="s")

    @functools.partial(
        pl.kernel,
        out_type=jax.ShapeDtypeStruct((n_rows, _IDX_W, embed_dim), jnp.float32),
        mesh=mesh,
        scratch_types=[
            pltpu.VMEM((rows_per_w, _IDX_W), jnp.int32),
            pltpu.VMEM((_K, _IDX_W, embed_dim), jnp.float32),
            pltpu.VMEM((_K, _IDX_W, embed_dim), jnp.float32),
            pltpu.SemaphoreType.DMA,
            pltpu.SemaphoreType.DMA,
        ],
        compiler_params=pltpu.CompilerParams(use_tc_tiling_on_sc=False),
    )
    def k(table_hbm, idx_hbm, out_hbm, idx_v, rows0, rows1, sem0, sem1):
        wid = lax.axis_index("s") * _NC + lax.axis_index("c")
        base = wid * rows_per_w
        rows = (rows0, rows1)
        sems = (sem0, sem1)

        pltpu.sync_copy(idx_hbm.at[pl.ds(base, rows_per_w)], idx_v)

        def fire(g, buf, sem):
            for j in range(_K):
                pltpu.async_copy(table_hbm.at[idx_v.at[g * _K + j]], buf.at[j], sem)

        fire(0, rows0, sem0)

        def step(g, carry):
            for b in range(2):

                @pl.when(g % 2 == b)
                def _():
                    buf, sem = rows[b], sems[b]
                    pltpu.make_async_copy(out_hbm.at[pl.ds(0, _K)], buf, sem).wait()

                    @pl.when(g < n_chunks - 1)
                    def _():
                        fire(g + 1, rows[1 - b], sems[1 - b])

                    pltpu.sync_copy(buf, out_hbm.at[pl.ds(base + g * _K, _K)])

            return carry

        lax.fori_loop(0, n_chunks, step, 0)

    return k(table, idx2d)


def kernel(inputs, embedding_matrix):
    batch, seq = inputs.shape
    vocab, embed_dim = embedding_matrix.shape
    n = batch * seq
    idx2d = inputs.astype(jnp.int32).reshape(n // _IDX_W, _IDX_W)
    out = _sc_gather(embedding_matrix, idx2d, embed_dim)
    return out.reshape(batch, seq, embed_dim)
